# baseline (speedup 1.0000x reference)
import functools
import jax
import jax.numpy as jnp
from jax import lax
from jax.experimental import pallas as pl
from jax.experimental.pallas import tpu as pltpu, tpu_sc as plsc

_CHUNK = 16


def _make_sc(B, S, D):
    info = plsc.get_sparse_core_info()
    NC, NS, L = info.num_cores, info.num_subcores, info.num_lanes
    NW = NC * NS
    s_per_w = S // NW
    n_chunks = s_per_w // _CHUNK
    vregs_per_row = D // L
    n_steps = n_chunks * B
    mesh = plsc.VectorSubcoreMesh(core_axis_name="c", subcore_axis_name="s")

    @functools.partial(
        pl.kernel,
        mesh=mesh,
        out_type=jax.ShapeDtypeStruct((B, S, D), jnp.float32),
        scratch_types=(
            [pltpu.VMEM((_CHUNK, D), jnp.float32)] * 6
            + [pltpu.SemaphoreType.DMA] * 10
        ),
    )
    def k(x_hbm, pos_hbm, out_hbm,
          pos_v0, pos_v1, x_v0, x_v1, x_v2, x_v3,
          sp0, sp1, sx0, sx1, sx2, sx3, so0, so1, so2, so3):
        wid = lax.axis_index("s") * NC + lax.axis_index("c")
        base = wid * s_per_w
        pos_bufs = [pos_v0, pos_v1]
        x_bufs = [x_v0, x_v1, x_v2, x_v3]
        sem_p, sem_x = [sp0, sp1], [sx0, sx1, sx2, sx3]
        sem_o = [so0, so1, so2, so3]

        x_handles = [None] * 4
        pos_handles = [None] * 2
        out_handles = [None] * 4

        def issue_x(s):
            t, b = divmod(s, B)
            xi = s % 4
            x_handles[xi] = pltpu.async_copy(
                x_hbm.at[b, pl.ds(base + t * _CHUNK, _CHUNK)],
                x_bufs[xi], sem_x[xi])

        def issue_pos(t):
            pp = t % 2
            pos_handles[pp] = pltpu.async_copy(
                pos_hbm.at[pl.ds(base + t * _CHUNK, _CHUNK)],
                pos_bufs[pp], sem_p[pp])

        issue_pos(0)
        for s0 in range(min(3, n_steps)):
            issue_x(s0)

        for s in range(n_steps):
            t, b = divmod(s, B)
            xi, pp = s % 4, t % 2

            x_handles[xi].wait()
            if b == 0:
                pos_handles[pp].wait()

            xb, pb = x_bufs[xi], pos_bufs[pp]

            @plsc.parallel_loop(0, _CHUNK * vregs_per_row, unroll=8)
            def _add(i, xb=xb, pb=pb):
                r = i // vregs_per_row
                sl = pl.ds((i % vregs_per_row) * L, L)
                xb[r, sl] = xb[r, sl] + pb[r, sl]

            out_handles[xi] = pltpu.async_copy(
                xb, out_hbm.at[b, pl.ds(base + t * _CHUNK, _CHUNK)],
                sem_o[xi])

            if b == 0 and t + 1 < n_chunks:
                issue_pos(t + 1)

            u = s + 3
            if u < n_steps:
                q = u % 4
                if out_handles[q] is not None:
                    out_handles[q].wait()
                    out_handles[q] = None
                issue_x(u)

        for q in range(4):
            if out_handles[q] is not None:
                out_handles[q].wait()

    return k


def kernel(x, pos_table):
    B, S, D = x.shape
    return _make_sc(B, S, D)(x, pos_table)
